# baseline (device time: 14258 ns/iter reference)
import jax
import jax.numpy as jnp
from jax import lax
from jax.experimental import pallas as pl
from jax.experimental.pallas import tpu as pltpu

NB = 8


def kernel(x, W, labels):
    T, D = x.shape
    V = W.shape[1]
    BV = V // NB
    labels2 = labels.reshape(1, T)

    def body(x_ref, w_ref, lab_ref, out_ref, s_acc, c_acc,
             send_buf, recv_buf, send_sem, recv_sem):
        step = pl.program_id(0)
        my_x = lax.axis_index("x")
        my_y = lax.axis_index("y")
        my_z = lax.axis_index("z")
        partner = (1 - my_x, my_y, my_z)

        @pl.when(step == 0)
        def _():
            barrier = pltpu.get_barrier_semaphore()
            pl.semaphore_signal(barrier, inc=1, device_id=partner,
                                device_id_type=pl.DeviceIdType.MESH)
            pl.semaphore_wait(barrier, 1)

        logits = lax.dot_general(
            w_ref[:, :].astype(jnp.bfloat16),
            x_ref[:, :].astype(jnp.bfloat16),
            dimension_numbers=(((0,), (1,)), ((), ())),
            preferred_element_type=jnp.float32,
        )
        s_part = jnp.sum(jnp.exp(logits), axis=0, keepdims=True)
        row = lax.broadcasted_iota(jnp.int32, (BV, T), 0)
        mask = row == (lab_ref[:, :] - my_x * V - step * BV)
        c_part = jnp.sum(jnp.where(mask, logits, 0.0),
                         axis=0, keepdims=True)

        @pl.when(step == 0)
        def _():
            s_acc[:, :] = s_part
            c_acc[:, :] = c_part

        @pl.when(step > 0)
        def _():
            s_acc[:, :] = s_acc[:, :] + s_part
            c_acc[:, :] = c_acc[:, :] + c_part

        @pl.when(step == NB - 1)
        def _():
            send_buf[0:1, :] = s_acc[:, :]
            send_buf[1:2, :] = c_acc[:, :]
            rdma = pltpu.make_async_remote_copy(
                src_ref=send_buf,
                dst_ref=recv_buf,
                send_sem=send_sem,
                recv_sem=recv_sem,
                device_id=partner,
                device_id_type=pl.DeviceIdType.MESH,
            )
            rdma.start()
            rdma.wait()

            s_tot = s_acc[:, :] + recv_buf[0:1, :]
            c_tot = c_acc[:, :] + recv_buf[1:2, :]
            out_ref[:, :] = jnp.log(s_tot) - c_tot

    out = pl.pallas_call(
        body,
        grid=(NB,),
        out_shape=jax.ShapeDtypeStruct((1, T), jnp.float32),
        in_specs=[
            pl.BlockSpec((T, D), lambda i: (0, 0)),
            pl.BlockSpec((D, BV), lambda i: (0, i)),
            pl.BlockSpec((1, T), lambda i: (0, 0)),
        ],
        out_specs=pl.BlockSpec((1, T), lambda i: (0, 0)),
        scratch_shapes=[
            pltpu.VMEM((1, T), jnp.float32),
            pltpu.VMEM((1, T), jnp.float32),
            pltpu.VMEM((2, T), jnp.float32),
            pltpu.VMEM((2, T), jnp.float32),
            pltpu.SemaphoreType.DMA,
            pltpu.SemaphoreType.DMA,
        ],
        compiler_params=pltpu.CompilerParams(collective_id=0),
    )(x, W, labels2)
    return out.reshape(T)


# device time: 12508 ns/iter; 1.1399x vs baseline; 1.1399x over previous
import jax
import jax.numpy as jnp
from jax import lax
from jax.experimental import pallas as pl
from jax.experimental.pallas import tpu as pltpu

NB = 8


def kernel(x, W, labels):
    T, D = x.shape
    V = W.shape[1]
    BV = V // NB
    labels2 = labels.reshape(1, T)

    def body(x_ref, w_ref, lab_ref, out_ref, s_acc, c_acc,
             send_buf, recv_buf, send_sem, recv_sem):
        step = pl.program_id(0)
        my_x = lax.axis_index("x")
        my_y = lax.axis_index("y")
        my_z = lax.axis_index("z")
        partner = (1 - my_x, my_y, my_z)

        @pl.when(step == 0)
        def _():
            barrier = pltpu.get_barrier_semaphore()
            pl.semaphore_signal(barrier, inc=1, device_id=partner,
                                device_id_type=pl.DeviceIdType.MESH)
            pl.semaphore_wait(barrier, 1)

        logits = lax.dot_general(
            w_ref[:, :].astype(jnp.bfloat16),
            x_ref[:, :].astype(jnp.bfloat16),
            dimension_numbers=(((0,), (1,)), ((), ())),
            preferred_element_type=jnp.float32,
        )
        s_part = jnp.sum(jnp.exp(logits), axis=0, keepdims=True)
        row = lax.broadcasted_iota(jnp.int32, (BV, T), 0)
        mask = row == (lab_ref[:, :] - my_x * V - step * BV)
        c_part = jnp.sum(jnp.where(mask, logits, 0.0),
                         axis=0, keepdims=True)

        @pl.when(step == 0)
        def _():
            s_acc[:, :] = s_part
            c_acc[:, :] = c_part

        @pl.when(step > 0)
        def _():
            s_acc[:, :] = s_acc[:, :] + s_part
            c_acc[:, :] = c_acc[:, :] + c_part

        @pl.when(step == NB - 1)
        def _():
            send_buf[0:1, :] = s_acc[:, :]
            send_buf[1:2, :] = c_acc[:, :]
            rdma = pltpu.make_async_remote_copy(
                src_ref=send_buf,
                dst_ref=recv_buf,
                send_sem=send_sem,
                recv_sem=recv_sem,
                device_id=partner,
                device_id_type=pl.DeviceIdType.MESH,
            )
            rdma.start()
            rdma.wait()

            s_tot = s_acc[:, :] + recv_buf[0:1, :]
            c_tot = c_acc[:, :] + recv_buf[1:2, :]
            out_ref[:, :] = jnp.log(s_tot) - c_tot

    out = pl.pallas_call(
        body,
        grid=(NB,),
        out_shape=jax.ShapeDtypeStruct((1, T), jnp.float32),
        in_specs=[
            pl.BlockSpec((T, D), lambda i: (0, 0)),
            pl.BlockSpec((D, BV), lambda i: (0, i)),
            pl.BlockSpec((1, T), lambda i: (0, 0)),
        ],
        out_specs=pl.BlockSpec((1, T), lambda i: (0, 0)),
        scratch_shapes=[
            pltpu.VMEM((1, T), jnp.float32),
            pltpu.VMEM((1, T), jnp.float32),
            pltpu.VMEM((2, T), jnp.float32),
            pltpu.VMEM((2, T), jnp.float32),
            pltpu.SemaphoreType.DMA,
            pltpu.SemaphoreType.DMA,
        ],
        compiler_params=pltpu.CompilerParams(collective_id=0),
    )(
        pltpu.with_memory_space_constraint(x, pltpu.MemorySpace.HBM),
        pltpu.with_memory_space_constraint(W, pltpu.MemorySpace.HBM),
        pltpu.with_memory_space_constraint(labels2, pltpu.MemorySpace.HBM),
    )
    return out.reshape(T)


# device time: 12057 ns/iter; 1.1825x vs baseline; 1.0374x over previous
import jax
import jax.numpy as jnp
from jax import lax
from jax.experimental import pallas as pl
from jax.experimental.pallas import tpu as pltpu

NC = 8


def kernel(x, W, labels):
    T, D = x.shape
    V = W.shape[1]
    BV = V // NC
    HALF = NC // 2
    labels2 = labels.reshape(1, T)

    def body(x_hbm, w_hbm, lab_hbm, out_hbm,
             x_v, w_v, lab_v, out_v, send_buf, recv_buf,
             w_sems, misc_sems, send_sems, recv_sems):
        my_x = lax.axis_index("x")
        my_y = lax.axis_index("y")
        my_z = lax.axis_index("z")
        partner = (1 - my_x, my_y, my_z)

        cx = pltpu.make_async_copy(x_hbm, x_v, misc_sems.at[0])
        cx.start()
        cl = pltpu.make_async_copy(lab_hbm, lab_v, misc_sems.at[1])
        cl.start()
        w_copies = []
        for i in range(NC):
            c = pltpu.make_async_copy(
                w_hbm.at[:, pl.ds(i * BV, BV)],
                w_v.at[:, pl.ds(i * BV, BV)],
                w_sems.at[i],
            )
            c.start()
            w_copies.append(c)

        barrier = pltpu.get_barrier_semaphore()
        pl.semaphore_signal(barrier, inc=1, device_id=partner,
                            device_id_type=pl.DeviceIdType.MESH)
        pl.semaphore_wait(barrier, 1)

        cx.wait()
        cl.wait()
        xb = x_v[:, :].astype(jnp.bfloat16)
        labrel = lab_v[:, :] - my_x * V
        row = lax.broadcasted_iota(jnp.int32, (BV, T), 0)

        def rdma_slot(slot):
            return pltpu.make_async_remote_copy(
                src_ref=send_buf.at[slot],
                dst_ref=recv_buf.at[slot],
                send_sem=send_sems.at[slot],
                recv_sem=recv_sems.at[slot],
                device_id=partner,
                device_id_type=pl.DeviceIdType.MESH,
            )

        def flush(slot, s_run, c_run):
            send_buf[slot, 0:1, :] = s_run
            send_buf[slot, 1:2, :] = c_run
            r = rdma_slot(slot)
            r.start()
            return r

        s_run = None
        c_run = None
        rdma0 = None
        for i in range(NC):
            w_copies[i].wait()
            logits = lax.dot_general(
                w_v[:, pl.ds(i * BV, BV)].astype(jnp.bfloat16), xb,
                dimension_numbers=(((0,), (1,)), ((), ())),
                preferred_element_type=jnp.float32,
            )
            s_part = jnp.sum(jnp.exp(logits), axis=0, keepdims=True)
            mask = row == (labrel - i * BV)
            c_part = jnp.sum(jnp.where(mask, logits, 0.0),
                             axis=0, keepdims=True)
            s_run = s_part if s_run is None else s_run + s_part
            c_run = c_part if c_run is None else c_run + c_part
            if i == HALF - 1:
                rdma0 = flush(0, s_run, c_run)
                s_run = None
                c_run = None

        rdma1 = flush(1, s_run, c_run)
        rdma0.wait()
        rdma1.wait()

        s_tot = (send_buf[0, 0:1, :] + send_buf[1, 0:1, :]
                 + recv_buf[0, 0:1, :] + recv_buf[1, 0:1, :])
        c_tot = (send_buf[0, 1:2, :] + send_buf[1, 1:2, :]
                 + recv_buf[0, 1:2, :] + recv_buf[1, 1:2, :])
        out_v[:, :] = jnp.log(s_tot) - c_tot
        co = pltpu.make_async_copy(out_v, out_hbm, misc_sems.at[2])
        co.start()
        co.wait()

    hbm = pltpu.MemorySpace.HBM
    out = pl.pallas_call(
        body,
        out_shape=jax.ShapeDtypeStruct((1, T), jnp.float32),
        in_specs=[
            pl.BlockSpec(memory_space=hbm),
            pl.BlockSpec(memory_space=hbm),
            pl.BlockSpec(memory_space=hbm),
        ],
        out_specs=pl.BlockSpec(memory_space=hbm),
        scratch_shapes=[
            pltpu.VMEM((T, D), jnp.float32),
            pltpu.VMEM((D, V), jnp.float32),
            pltpu.VMEM((1, T), jnp.int32),
            pltpu.VMEM((1, T), jnp.float32),
            pltpu.VMEM((2, 2, T), jnp.float32),
            pltpu.VMEM((2, 2, T), jnp.float32),
            pltpu.SemaphoreType.DMA((NC,)),
            pltpu.SemaphoreType.DMA((3,)),
            pltpu.SemaphoreType.DMA((2,)),
            pltpu.SemaphoreType.DMA((2,)),
        ],
        compiler_params=pltpu.CompilerParams(collective_id=0),
    )(
        pltpu.with_memory_space_constraint(x, hbm),
        pltpu.with_memory_space_constraint(W, hbm),
        pltpu.with_memory_space_constraint(labels2, hbm),
    )
    return out.reshape(T)


# device time: 9942 ns/iter; 1.4341x vs baseline; 1.2127x over previous
import jax
import jax.numpy as jnp
from jax import lax
from jax.experimental import pallas as pl
from jax.experimental.pallas import tpu as pltpu

NC = 2


def kernel(x, W, labels):
    T, D = x.shape
    V = W.shape[1]
    BV = V // NC
    labels2 = labels.reshape(1, T)

    def body(x_hbm, w_hbm, lab_hbm, out_hbm,
             x_v, w0_v, w1_v, lab_v, out_v, send_buf, recv_buf,
             w_sems, misc_sems, send_sems, recv_sems):
        my_x = lax.axis_index("x")
        my_y = lax.axis_index("y")
        my_z = lax.axis_index("z")
        partner = (1 - my_x, my_y, my_z)
        w_bufs = [w0_v, w1_v]

        cx = pltpu.make_async_copy(x_hbm, x_v, misc_sems.at[0])
        cx.start()
        cl = pltpu.make_async_copy(lab_hbm, lab_v, misc_sems.at[1])
        cl.start()
        w_copies = []
        for i in range(NC):
            c = pltpu.make_async_copy(
                w_hbm.at[:, pl.ds(i * BV, BV)], w_bufs[i], w_sems.at[i])
            c.start()
            w_copies.append(c)

        cx.wait()
        cl.wait()
        xb = x_v[:, :].astype(jnp.bfloat16)
        labrel = lab_v[:, :] - my_x * V
        row = lax.broadcasted_iota(jnp.int32, (BV, T), 0)

        def part(i):
            logits = lax.dot_general(
                w_bufs[i][:, :].astype(jnp.bfloat16), xb,
                dimension_numbers=(((0,), (1,)), ((), ())),
                preferred_element_type=jnp.float32,
            )
            s = jnp.sum(jnp.exp(logits), axis=0, keepdims=True)
            mask = row == (labrel - i * BV)
            c = jnp.sum(jnp.where(mask, logits, 0.0), axis=0, keepdims=True)
            return s, c

        def flush(slot, s, c):
            send_buf[slot, 0:1, :] = s
            send_buf[slot, 1:2, :] = c
            r = pltpu.make_async_remote_copy(
                src_ref=send_buf.at[slot],
                dst_ref=recv_buf.at[slot],
                send_sem=send_sems.at[slot],
                recv_sem=recv_sems.at[slot],
                device_id=partner,
                device_id_type=pl.DeviceIdType.MESH,
            )
            r.start()
            return r

        w_copies[0].wait()
        s0, c0 = part(0)

        barrier = pltpu.get_barrier_semaphore()
        pl.semaphore_signal(barrier, inc=1, device_id=partner,
                            device_id_type=pl.DeviceIdType.MESH)
        pl.semaphore_wait(barrier, 1)

        rdma0 = flush(0, s0, c0)

        w_copies[1].wait()
        s1, c1 = part(1)
        rdma1 = flush(1, s1, c1)

        rdma0.wait()
        rdma1.wait()
        s_tot = s0 + s1 + recv_buf[0, 0:1, :] + recv_buf[1, 0:1, :]
        c_tot = c0 + c1 + recv_buf[0, 1:2, :] + recv_buf[1, 1:2, :]
        out_v[:, :] = jnp.log(s_tot) - c_tot
        co = pltpu.make_async_copy(out_v, out_hbm, misc_sems.at[2])
        co.start()
        co.wait()

    hbm = pltpu.MemorySpace.HBM
    out = pl.pallas_call(
        body,
        out_shape=jax.ShapeDtypeStruct((1, T), jnp.float32),
        in_specs=[
            pl.BlockSpec(memory_space=hbm),
            pl.BlockSpec(memory_space=hbm),
            pl.BlockSpec(memory_space=hbm),
        ],
        out_specs=pl.BlockSpec(memory_space=hbm),
        scratch_shapes=[
            pltpu.VMEM((T, D), jnp.float32),
            pltpu.VMEM((D, V // NC), jnp.float32),
            pltpu.VMEM((D, V // NC), jnp.float32),
            pltpu.VMEM((1, T), jnp.int32),
            pltpu.VMEM((1, T), jnp.float32),
            pltpu.VMEM((2, 2, T), jnp.float32),
            pltpu.VMEM((2, 2, T), jnp.float32),
            pltpu.SemaphoreType.DMA((NC,)),
            pltpu.SemaphoreType.DMA((3,)),
            pltpu.SemaphoreType.DMA((2,)),
            pltpu.SemaphoreType.DMA((2,)),
        ],
        compiler_params=pltpu.CompilerParams(collective_id=0),
    )(
        pltpu.with_memory_space_constraint(x, hbm),
        pltpu.with_memory_space_constraint(W, hbm),
        pltpu.with_memory_space_constraint(labels2, hbm),
    )
    return out.reshape(T)
